# Initial kernel scaffold; baseline (speedup 1.0000x reference)
import jax
import jax.numpy as jnp
from jax.experimental import pallas as pl


def kernel(x, input_pts, cv2_weight, cv2_centers, cv2_l1_w, cv2_l1_b, cv2_l2_w, cv2_l2_b, cv2_l3_w, cv2_l3_b, cv3_weight, cv3_centers, cv3_l1_w, cv3_l1_b, cv3_l2_w, cv3_l2_b, cv3_l3_w, cv3_l3_b, cv4_weight, cv4_centers, cv4_l1_w, cv4_l1_b, cv4_l2_w, cv4_l2_b, cv4_l3_w, cv4_l3_b, bn2_g, bn2_b, bn3_g, bn3_b):
    raise NotImplementedError("write your pallas kernel here")


# speedup vs baseline: 10.6702x; 10.6702x over previous
import functools

import jax
import jax.numpy as jnp
import numpy as np
from jax.experimental import pallas as pl


def _ptconv_body(k, dim, BQ, N, Kc, qpts_ref, ptsT_ref, pf_ref,
                 w1_ref, b1_ref, w2_ref, b2_ref, w3_ref, b3_ref,
                 cflat_ref, rexp_ref, texp_ref, wmat_ref, out_ref):
    q = qpts_ref[0]
    pt = ptsT_ref[0]
    qq = jnp.sum(q * q, axis=1, keepdims=True)
    pp = jnp.sum(pt * pt, axis=0, keepdims=True)
    qp = jnp.dot(q, pt, preferred_element_type=jnp.float32)
    D = qq + pp - 2.0 * qp

    iota = jax.lax.broadcasted_iota(jnp.int32, (BQ, N), 1)
    pf = pf_ref[0]
    C = pf.shape[-1] // 3
    rel_l, ft_l = [], []
    for _ in range(k):
        m = jnp.min(D, axis=1, keepdims=True)
        idx = jnp.min(jnp.where(D == m, iota, N), axis=1, keepdims=True)
        oh = iota == idx
        D = jnp.where(oh, jnp.float32(float("inf")), D)
        g3 = jnp.dot(oh.astype(jnp.float32), pf,
                     preferred_element_type=jnp.float32)
        g = g3[:, :C] + g3[:, C:2 * C] + g3[:, 2 * C:]
        rel_l.append(g[:, :dim] - q)
        ft_l.append(g[:, dim:])
    rel = jnp.concatenate(rel_l, axis=0)
    ft = jnp.concatenate(ft_l, axis=0)

    dists = jnp.concatenate(
        [rel[:, dd:dd + 1] - cflat_ref[:, dd * Kc:(dd + 1) * Kc]
         for dd in range(dim)], axis=1)
    h = jnp.maximum(jnp.dot(dists, w1_ref[...],
                            preferred_element_type=jnp.float32) + b1_ref[...], 0.0)
    h = jnp.maximum(jnp.dot(h, w2_ref[...],
                            preferred_element_type=jnp.float32) + b2_ref[...], 0.0)
    h = jnp.maximum(jnp.dot(h, w3_ref[...],
                            preferred_element_type=jnp.float32) + b3_ref[...], 0.0)
    fexp = jnp.dot(ft, rexp_ref[...], preferred_element_type=jnp.float32)
    dtil = jnp.dot(h, texp_ref[...], preferred_element_type=jnp.float32)
    prod = fexp * dtil
    CK = prod.shape[-1]
    f = prod.reshape(k, BQ, CK).sum(axis=0)
    ok = jnp.dot(f, wmat_ref[...],
                 preferred_element_type=jnp.float32)
    out_ref[0] = ok * (1.0 / k)


def _ptconv(features, pts_in, k, out_num,
            weight, centers, l1w, l1b, l2w, l2b, l3w, l3b):
    B, N, dim = pts_in.shape
    Cin = features.shape[-1]
    Kc = centers.shape[1]
    Cout = weight.shape[-1]
    BQ = min(128, out_num)

    queries = pts_in[:, :out_num]
    ptsT = jnp.swapaxes(pts_in, 1, 2)
    pf0 = jnp.concatenate([pts_in, features], axis=-1)
    p1 = jax.lax.reduce_precision(pf0, 8, 7)
    r = pf0 - p1
    p2 = jax.lax.reduce_precision(r, 8, 7)
    p3 = r - p2
    pf = jnp.concatenate([p1, p2, p3], axis=-1)
    w1, w2, w3 = l1w.T, l2w.T, l3w.T
    b1, b2, b3 = l1b.reshape(1, -1), l2b.reshape(1, -1), l3b.reshape(1, -1)
    cflat = centers.reshape(1, dim * Kc)
    rexp = jnp.asarray(np.kron(np.eye(Cin, dtype=np.float32),
                               np.ones((1, Kc), np.float32)))
    texp = jnp.asarray(np.tile(np.eye(Kc, dtype=np.float32), (1, Cin)))
    wmat = weight.reshape(Cin * Kc, Cout)

    grid = (B, out_num // BQ)
    full2d = lambda a: pl.BlockSpec(a.shape, lambda b, qi: (0, 0))
    out = pl.pallas_call(
        functools.partial(_ptconv_body, k, dim, BQ, N, Kc),
        grid=grid,
        in_specs=[
            pl.BlockSpec((1, BQ, dim), lambda b, qi: (b, qi, 0)),
            pl.BlockSpec((1, dim, N), lambda b, qi: (b, 0, 0)),
            pl.BlockSpec((1, N, 3 * (dim + Cin)), lambda b, qi: (b, 0, 0)),
            full2d(w1), full2d(b1), full2d(w2), full2d(b2),
            full2d(w3), full2d(b3), full2d(cflat),
            full2d(rexp), full2d(texp), full2d(wmat),
        ],
        out_specs=pl.BlockSpec((1, BQ, Cout), lambda b, qi: (b, qi, 0)),
        out_shape=jax.ShapeDtypeStruct((B, out_num, Cout), jnp.float32),
    )(queries, ptsT, pf, w1, b1, w2, b2, w3, b3, cflat, rexp, texp, wmat)
    return out, queries


def _bn_relu_body(x_ref, g_ref, b_ref, o_ref):
    xv = x_ref[...]
    m = jnp.mean(xv, axis=0, keepdims=True)
    v = jnp.mean((xv - m) ** 2, axis=0, keepdims=True)
    o_ref[...] = jnp.maximum(
        (xv - m) / jnp.sqrt(v + 1e-5) * g_ref[...] + b_ref[...], 0.0)


def _bn_relu(x, g, b):
    B, Q, C = x.shape
    out = pl.pallas_call(
        _bn_relu_body,
        out_shape=jax.ShapeDtypeStruct((B * Q, C), jnp.float32),
    )(x.reshape(B * Q, C), g.reshape(1, C), b.reshape(1, C))
    return out.reshape(B, Q, C)


def kernel(x, input_pts, cv2_weight, cv2_centers, cv2_l1_w, cv2_l1_b,
           cv2_l2_w, cv2_l2_b, cv2_l3_w, cv2_l3_b, cv3_weight, cv3_centers,
           cv3_l1_w, cv3_l1_b, cv3_l2_w, cv3_l2_b, cv3_l3_w, cv3_l3_b,
           cv4_weight, cv4_centers, cv4_l1_w, cv4_l1_b, cv4_l2_w, cv4_l2_b,
           cv4_l3_w, cv4_l3_b, bn2_g, bn2_b, bn3_g, bn3_b):
    x2, pts2 = _ptconv(x, input_pts, 16, 1024, cv2_weight, cv2_centers,
                       cv2_l1_w, cv2_l1_b, cv2_l2_w, cv2_l2_b, cv2_l3_w, cv2_l3_b)
    x2 = _bn_relu(x2, bn2_g, bn2_b)
    x3, pts3 = _ptconv(x2, pts2, 16, 256, cv3_weight, cv3_centers,
                       cv3_l1_w, cv3_l1_b, cv3_l2_w, cv3_l2_b, cv3_l3_w, cv3_l3_b)
    x3 = _bn_relu(x3, bn3_g, bn3_b)
    x4, pts4 = _ptconv(x3, pts3, 8, 64, cv4_weight, cv4_centers,
                       cv4_l1_w, cv4_l1_b, cv4_l2_w, cv4_l2_b, cv4_l3_w, cv4_l3_b)
    return (x4, pts4)
